# baseline (speedup 1.0000x reference)
import functools
import math

import jax
import jax.numpy as jnp
from jax import lax
from jax.experimental import pallas as pl
from jax.experimental.pallas import tpu as pltpu
from jax.experimental.pallas import tpu_sc as plsc

D_M = 1024
K_REP = 6
TEMP = 0.1
N_HEADS = 8
HEAD_DIM = D_M // N_HEADS
TB = 512
B_SZ = 2


def _dot_t(x, w):
    return lax.dot_general(x, w, (((1,), (1,)), ((), ())),
                           preferred_element_type=jnp.float32)


def _fused_kernel(f_blk, qe, mu_w1, mu_b1, mu_g1, mu_be1, mu_w2, mu_b2,
                  sg_w1, sg_b1, sg_g1, sg_be1, sg_w2, sg_b2,
                  wq, bq, wk,
                  idx_ref, comb_ref, gidx_ref, ell_ref, mu_out, isg_out,
                  mu_sc, isg_sc, u_sc, dist_sc, hl_sc):
    B = qe.shape[0]
    T = dist_sc.shape[1]
    NT = T // TB
    i = pl.program_id(0)

    @pl.when(i == 0)
    def _stage_q():
        def layernorm(x, g, b):
            m = jnp.mean(x, axis=-1, keepdims=True)
            v = jnp.mean((x - m) ** 2, axis=-1, keepdims=True)
            return (x - m) / jnp.sqrt(v + 1e-5) * g + b

        def mlp(x, w1, b1, g, be, w2, b2):
            h = _dot_t(x, w1[...]) + b1[...][None, :]
            h = jnp.maximum(layernorm(h, g[...][None, :], be[...][None, :]),
                            0.0)
            return _dot_t(h, w2[...]) + b2[...][None, :]

        x = qe[...]
        mu = mlp(x, mu_w1, mu_b1, mu_g1, mu_be1, mu_w2, mu_b2)
        sg_pre = mlp(x, sg_w1, sg_b1, sg_g1, sg_be1, sg_w2, sg_b2)
        sigma = jnp.maximum(sg_pre, 0.0) + jnp.log1p(jnp.exp(-jnp.abs(sg_pre)))
        sigma = sigma + 1e-6
        mu_sc[...] = mu
        isg_sc[...] = 1.0 / sigma

        q = _dot_t(mu, wq[...]) + bq[...][None, :]
        nrow = B * N_HEADS
        qb = jnp.broadcast_to(q[:, None, :], (B, N_HEADS, D_M)).reshape(
            nrow, D_M)
        col_h = lax.broadcasted_iota(jnp.int32, (nrow, D_M), 1) // HEAD_DIM
        row_h = lax.broadcasted_iota(jnp.int32, (nrow, D_M), 0) % N_HEADS
        q8 = jnp.where(col_h == row_h, qb, 0.0)
        scale = 1.0 / math.sqrt(HEAD_DIM)
        u_sc[...] = jnp.dot(q8, wk[...],
                            preferred_element_type=jnp.float32) * scale

    @pl.when((i >= 1) & (i <= NT))
    def _stage_stream():
        t0 = (i - 1) * TB
        for b in range(B):
            f = f_blk[b]
            cen = f - mu_sc[b][None, :]
            dist_sc[b, pl.ds(t0, TB)] = jnp.sum(
                cen * cen * isg_sc[b][None, :], axis=1)
            hl = lax.dot_general(
                u_sc[pl.ds(b * N_HEADS, N_HEADS), :], f,
                (((1,), (1,)), ((), ())),
                preferred_element_type=jnp.float32)
            hl_sc[pl.ds(b * N_HEADS, N_HEADS), pl.ds(t0, TB)] = hl

    @pl.when(i == NT + 1)
    def _stage_final():
        pos = lax.broadcasted_iota(jnp.int32, (B, T), 1)
        posf = pos.astype(jnp.float32)
        dist = dist_sc[...]

        bits = lax.bitcast_convert_type(dist, jnp.int32)
        bp = bits.reshape(B * T // 128, 128)
        rank = (T - 1) // 2
        rows = T // 128
        med_s = []
        for b in range(B):
            bpb = bp[b * rows:(b + 1) * rows]

            def count_lt(cand, _bpb=bpb):
                return jnp.sum((_bpb < cand).astype(jnp.int32))

            m = jnp.int32(0)
            b30 = jnp.int32(1 << 30)
            m = jnp.where(count_lt(m | b30) <= rank, m | b30, m)
            for k in range(29, 0, -2):
                hi = jnp.int32(1 << k)
                lo = jnp.int32(1 << (k - 1))
                c_lo = count_lt(m | lo) <= rank
                c_hi = count_lt(m | hi) <= rank
                c_both = count_lt(m | hi | lo) <= rank
                m = m | jnp.where(c_hi, hi, jnp.int32(0))
                m = m | jnp.where(jnp.where(c_hi, c_both, c_lo), lo,
                                  jnp.int32(0))
            med_s.append(lax.bitcast_convert_type(m, jnp.float32))
        row_id = lax.broadcasted_iota(jnp.int32, (B, 1), 0)
        med = jnp.where(row_id == 0, med_s[0], med_s[1])

        dw = jnp.exp(-jnp.abs(dist - med) / TEMP)

        hl = hl_sc[...]
        mx = jnp.max(hl, axis=1, keepdims=True)
        e = jnp.exp(hl - mx)
        attn = e / jnp.sum(e, axis=1, keepdims=True)
        attn_mean = jnp.mean(attn.reshape(B, N_HEADS, T), axis=1)

        w = dw * attn_mean
        comb_ref[...] = w

        def first_argmax(v):
            mv = jnp.max(v, axis=1, keepdims=True)
            return jnp.min(jnp.where(v == mv, pos, T), axis=1, keepdims=True)

        idxv = [first_argmax(w)]
        min_dist = jnp.abs(posf - idxv[0].astype(jnp.float32))
        sel = pos == idxv[0]
        for _ in range(K_REP - 1):
            score = jnp.where(sel, -jnp.inf, min_dist * w)
            nxt = first_argmax(score)
            idxv.append(nxt)
            min_dist = jnp.minimum(min_dist,
                                   jnp.abs(posf - nxt.astype(jnp.float32)))
            sel = sel | (pos == nxt)

        rd = [jnp.sum(jnp.where(pos == idxv[k], dist, 0.0), axis=1,
                      keepdims=True) for k in range(K_REP)]
        t_rank = (K_REP - 1) // 2
        target = jnp.zeros((B, 1), jnp.float32)
        for a in range(K_REP):
            r_a = jnp.zeros((B, 1), jnp.int32)
            for j in range(K_REP):
                if j == a:
                    continue
                less = rd[j] < rd[a]
                if j < a:
                    less = less | (rd[j] == rd[a])
                r_a = r_a + less.astype(jnp.int32)
            target = target + jnp.where(r_a == t_rank, rd[a], 0.0)
        ell_sum = 0.0
        for k in range(K_REP):
            ell_sum = ell_sum + jnp.sum((rd[k] - target) ** 2)

        bsel = [lax.broadcasted_iota(jnp.int32, (B, 1), 0) == b
                for b in range(B)]
        lane16 = lax.broadcasted_iota(jnp.int32, (16,), 0)
        gidx = jnp.zeros((16,), jnp.int32)
        for b in range(B):
            for k in range(K_REP):
                s = jnp.sum(jnp.where(bsel[b], idxv[k], 0))
                idx_ref[b, k] = s
                gidx = jnp.where(lane16 == b * K_REP + k, s + b * T, gidx)
        gidx_ref[...] = gidx
        ell = ell_sum / (B * K_REP)
        ell_ref[...] = jnp.where(lane16 == 0, ell, 0.0)
        mu_out[...] = mu_sc[...]
        isg_out[...] = isg_sc[...]


def _sc_body(feats_hbm, gidx_hbm, mu_hbm, isg_hbm, ell_hbm,
             rep_out, loss_out,
             idx_v, rows_v, mu_v, isg_v, loss_v, sem):
    cid = lax.axis_index("c")
    sid = lax.axis_index("s")
    is_lead = (cid == 0) & (sid == 0)
    nk = B_SZ * K_REP

    pltpu.sync_copy(gidx_hbm, idx_v)
    pltpu.async_copy(feats_hbm.at[idx_v], rows_v, sem).wait()
    pltpu.sync_copy(mu_hbm, mu_v)
    pltpu.sync_copy(isg_hbm, isg_v)

    nch = D_M // 16
    zero = jnp.zeros((16,), jnp.float32)
    pairs = [(b, i2, j2) for b in range(B_SZ)
             for i2 in range(K_REP) for j2 in range(i2 + 1, K_REP)]

    con = zero
    pacc = [zero for _ in pairs]
    for j in range(nch):
        sl = pl.ds(j * 16, 16)
        row = [[rows_v[b * K_REP + r, sl] for r in range(K_REP)]
               for b in range(B_SZ)]
        for b in range(B_SZ):
            mub = mu_v[b, sl]
            isb = isg_v[b, sl]
            for r in range(K_REP):
                cen = row[b][r] - mub
                con = con + cen * cen * isb
        for p, (b, i2, j2) in enumerate(pairs):
            pacc[p] = pacc[p] + row[b][i2] * row[b][j2]
    res = (con,) + tuple(pacc)

    def sum16(v):
        s = v[0]
        for q in range(1, 16):
            s = s + v[q]
        return s

    con_s = sum16(res[0])
    div_s = jnp.float32(0.0)
    for p in range(len(pairs)):
        d = sum16(res[1 + p])
        div_s = div_s + d * d
    pltpu.sync_copy(ell_hbm, loss_v)
    ell_s = loss_v[...][0]
    loss = (ell_s + 0.1 * (con_s * (1.0 / nk))
            + 0.05 * (div_s * (1.0 / (B_SZ * K_REP * K_REP))))
    loss_v[...] = jnp.broadcast_to(loss, (16,))

    @pl.when(is_lead)
    def _():
        pltpu.sync_copy(rows_v, rep_out)
        pltpu.sync_copy(loss_v, loss_out)


def _sc_gather_loss(feats_flat, gidx, mu_q, inv_sigma, ell):
    nk = B_SZ * K_REP
    run = pl.kernel(
        _sc_body,
        mesh=plsc.VectorSubcoreMesh(core_axis_name="c", subcore_axis_name="s"),
        out_type=[
            jax.ShapeDtypeStruct((16, D_M), jnp.float32),
            jax.ShapeDtypeStruct((16,), jnp.float32),
        ],
        scratch_types=[
            pltpu.VMEM((16,), jnp.int32),
            pltpu.VMEM((16, D_M), jnp.float32),
            pltpu.VMEM((B_SZ, D_M), jnp.float32),
            pltpu.VMEM((B_SZ, D_M), jnp.float32),
            pltpu.VMEM((16,), jnp.float32),
            pltpu.SemaphoreType.DMA,
        ],
    )
    return run(feats_flat, gidx, mu_q, inv_sigma, ell)


def kernel(video_features, query_embedding, mu_w1, mu_b1, mu_g1, mu_be1,
           mu_w2, mu_b2, sg_w1, sg_b1, sg_g1, sg_be1, sg_w2, sg_b2,
           in_proj_w, in_proj_b, out_w, out_b):
    B, T, D = video_features.shape
    NT = T // TB

    def const2(_):
        return (0, 0)

    def const1(_):
        return (0,)

    full2 = pl.BlockSpec((D, D), const2)
    full1 = pl.BlockSpec((D,), const1)

    indices, combined, gidx, ell, mu_q, inv_sigma = pl.pallas_call(
        _fused_kernel,
        grid=(NT + 2,),
        in_specs=[
            pl.BlockSpec((B, TB, D),
                         lambda i: (0, jnp.clip(i - 1, 0, NT - 1), 0)),
            pl.BlockSpec((B, D), const2),
            full2, full1, full1, full1, full2, full1,
            full2, full1, full1, full1, full2, full1,
            pl.BlockSpec((D, D), lambda i: (0, 0)),
            pl.BlockSpec((D,), lambda i: (0,)),
            pl.BlockSpec((D, D), lambda i: (1, 0)),
        ],
        out_specs=(
            pl.BlockSpec(memory_space=pltpu.SMEM),
            pl.BlockSpec((B, T), const2),
            pl.BlockSpec((16,), const1),
            pl.BlockSpec((16,), const1),
            pl.BlockSpec((B, D), const2),
            pl.BlockSpec((B, D), const2),
        ),
        out_shape=(
            jax.ShapeDtypeStruct((B, K_REP), jnp.int32),
            jax.ShapeDtypeStruct((B, T), jnp.float32),
            jax.ShapeDtypeStruct((16,), jnp.int32),
            jax.ShapeDtypeStruct((16,), jnp.float32),
            jax.ShapeDtypeStruct((B, D), jnp.float32),
            jax.ShapeDtypeStruct((B, D), jnp.float32),
        ),
        scratch_shapes=[
            pltpu.VMEM((B, D), jnp.float32),
            pltpu.VMEM((B, D), jnp.float32),
            pltpu.VMEM((B * N_HEADS, D), jnp.float32),
            pltpu.VMEM((B, T), jnp.float32),
            pltpu.VMEM((B * N_HEADS, T), jnp.float32),
        ],
    )(video_features, query_embedding, mu_w1, mu_b1, mu_g1, mu_be1,
      mu_w2, mu_b2, sg_w1, sg_b1, sg_g1, sg_be1, sg_w2, sg_b2,
      in_proj_w, in_proj_b, in_proj_w)

    feats_flat = video_features.reshape(B * T, D)
    rep_flat, loss_vec = _sc_gather_loss(feats_flat, gidx, mu_q, inv_sigma,
                                         ell)
    rep = rep_flat[:B * K_REP].reshape(B, K_REP, D)
    return rep, loss_vec[0], indices, combined


# speedup vs baseline: 1.0554x; 1.0554x over previous
import functools
import math

import jax
import jax.numpy as jnp
from jax import lax
from jax.experimental import pallas as pl
from jax.experimental.pallas import tpu as pltpu
from jax.experimental.pallas import tpu_sc as plsc

D_M = 1024
K_REP = 6
TEMP = 0.1
N_HEADS = 8
HEAD_DIM = D_M // N_HEADS
TB = 512
B_SZ = 2


def _dot_t(x, w):
    return lax.dot_general(x, w, (((1,), (1,)), ((), ())),
                           preferred_element_type=jnp.float32)


def _fused_kernel(f_blk, qe, mu_w1, mu_b1, mu_g1, mu_be1, mu_w2, mu_b2,
                  sg_w1, sg_b1, sg_g1, sg_be1, sg_w2, sg_b2,
                  wq, bq, wk,
                  idx_ref, comb_ref, gidx_ref, ell_ref, mu_out, isg_out,
                  mu_sc, isg_sc, u_sc, dist_sc, hl_sc):
    B = qe.shape[0]
    T = dist_sc.shape[1]
    NT = T // TB
    i = pl.program_id(0)

    @pl.when(i == 0)
    def _stage_q():
        def layernorm(x, g, b):
            m = jnp.mean(x, axis=-1, keepdims=True)
            v = jnp.mean((x - m) ** 2, axis=-1, keepdims=True)
            return (x - m) / jnp.sqrt(v + 1e-5) * g + b

        def mlp(x, w1, b1, g, be, w2, b2):
            h = _dot_t(x, w1[...]) + b1[...][None, :]
            h = jnp.maximum(layernorm(h, g[...][None, :], be[...][None, :]),
                            0.0)
            return _dot_t(h, w2[...]) + b2[...][None, :]

        x = qe[...]
        mu = mlp(x, mu_w1, mu_b1, mu_g1, mu_be1, mu_w2, mu_b2)
        sg_pre = mlp(x, sg_w1, sg_b1, sg_g1, sg_be1, sg_w2, sg_b2)
        sigma = jnp.maximum(sg_pre, 0.0) + jnp.log1p(jnp.exp(-jnp.abs(sg_pre)))
        sigma = sigma + 1e-6
        mu_sc[...] = mu
        isg_sc[...] = 1.0 / sigma

        q = _dot_t(mu, wq[...]) + bq[...][None, :]
        nrow = B * N_HEADS
        qb = jnp.broadcast_to(q[:, None, :], (B, N_HEADS, D_M)).reshape(
            nrow, D_M)
        col_h = lax.broadcasted_iota(jnp.int32, (nrow, D_M), 1) // HEAD_DIM
        row_h = lax.broadcasted_iota(jnp.int32, (nrow, D_M), 0) % N_HEADS
        q8 = jnp.where(col_h == row_h, qb, 0.0)
        scale = 1.0 / math.sqrt(HEAD_DIM)
        u_sc[...] = jnp.dot(q8, wk[...],
                            preferred_element_type=jnp.float32) * scale

    @pl.when((i >= 1) & (i <= NT))
    def _stage_stream():
        t0 = (i - 1) * TB
        for b in range(B):
            f = f_blk[b]
            cen = f - mu_sc[b][None, :]
            dist_sc[b, pl.ds(t0, TB)] = jnp.sum(
                cen * cen * isg_sc[b][None, :], axis=1)
            hl = lax.dot_general(
                u_sc[pl.ds(b * N_HEADS, N_HEADS), :], f,
                (((1,), (1,)), ((), ())),
                preferred_element_type=jnp.float32)
            hl_sc[pl.ds(b * N_HEADS, N_HEADS), pl.ds(t0, TB)] = hl

    @pl.when(i == NT + 1)
    def _stage_final():
        pos = lax.broadcasted_iota(jnp.int32, (B, T), 1)
        posf = pos.astype(jnp.float32)
        dist = dist_sc[...]

        bits = lax.bitcast_convert_type(dist, jnp.int32)
        bp = bits.reshape(B * T // 128, 128)
        rank = (T - 1) // 2
        rows = T // 128
        med_s = []
        for b in range(B):
            bpb = bp[b * rows:(b + 1) * rows]

            def count_lt(cand, _bpb=bpb):
                return jnp.sum((_bpb < cand).astype(jnp.int32))

            m = jnp.int32(0)
            b30 = jnp.int32(1 << 30)
            m = jnp.where(count_lt(m | b30) <= rank, m | b30, m)
            for k in range(29, 0, -2):
                hi = jnp.int32(1 << k)
                lo = jnp.int32(1 << (k - 1))
                c_lo = count_lt(m | lo) <= rank
                c_hi = count_lt(m | hi) <= rank
                c_both = count_lt(m | hi | lo) <= rank
                m = m | jnp.where(c_hi, hi, jnp.int32(0))
                m = m | jnp.where(jnp.where(c_hi, c_both, c_lo), lo,
                                  jnp.int32(0))
            med_s.append(lax.bitcast_convert_type(m, jnp.float32))
        row_id = lax.broadcasted_iota(jnp.int32, (B, 1), 0)
        med = jnp.where(row_id == 0, med_s[0], med_s[1])

        dw = jnp.exp(-jnp.abs(dist - med) / TEMP)

        hl = hl_sc[...]
        mx = jnp.max(hl, axis=1, keepdims=True)
        e = jnp.exp(hl - mx)
        attn = e / jnp.sum(e, axis=1, keepdims=True)
        attn_mean = jnp.mean(attn.reshape(B, N_HEADS, T), axis=1)

        w = dw * attn_mean
        comb_ref[...] = w

        def first_argmax(v):
            mv = jnp.max(v, axis=1, keepdims=True)
            return jnp.min(jnp.where(v == mv, pos, T), axis=1, keepdims=True)

        idxv = [first_argmax(w)]
        min_dist = jnp.abs(posf - idxv[0].astype(jnp.float32))
        sel = pos == idxv[0]
        for _ in range(K_REP - 1):
            score = jnp.where(sel, -jnp.inf, min_dist * w)
            nxt = first_argmax(score)
            idxv.append(nxt)
            min_dist = jnp.minimum(min_dist,
                                   jnp.abs(posf - nxt.astype(jnp.float32)))
            sel = sel | (pos == nxt)

        rd = [jnp.sum(jnp.where(pos == idxv[k], dist, 0.0), axis=1,
                      keepdims=True) for k in range(K_REP)]
        t_rank = (K_REP - 1) // 2
        target = jnp.zeros((B, 1), jnp.float32)
        for a in range(K_REP):
            r_a = jnp.zeros((B, 1), jnp.int32)
            for j in range(K_REP):
                if j == a:
                    continue
                less = rd[j] < rd[a]
                if j < a:
                    less = less | (rd[j] == rd[a])
                r_a = r_a + less.astype(jnp.int32)
            target = target + jnp.where(r_a == t_rank, rd[a], 0.0)
        ell_sum = 0.0
        for k in range(K_REP):
            ell_sum = ell_sum + jnp.sum((rd[k] - target) ** 2)

        bsel = [lax.broadcasted_iota(jnp.int32, (B, 1), 0) == b
                for b in range(B)]
        lane16 = lax.broadcasted_iota(jnp.int32, (16,), 0)
        gidx = jnp.zeros((16,), jnp.int32)
        for b in range(B):
            for k in range(K_REP):
                s = jnp.sum(jnp.where(bsel[b], idxv[k], 0))
                idx_ref[b, k] = s
                gidx = jnp.where(lane16 == b * K_REP + k, s + b * T, gidx)
        gidx_ref[...] = gidx
        ell = ell_sum / (B * K_REP)
        ell_ref[...] = jnp.where(lane16 == 0, ell, 0.0)
        mu_out[...] = mu_sc[...]
        isg_out[...] = isg_sc[...]


def _sc_body(feats_hbm, gidx_hbm, mu_hbm, isg_hbm, ell_hbm,
             rep_out, loss_out,
             idx_v, rows_v, mu_v, isg_v, loss_v, sem):
    cid = lax.axis_index("c")
    sid = lax.axis_index("s")
    is_lead = (cid == 0) & (sid == 0)
    nk = B_SZ * K_REP

    pltpu.sync_copy(gidx_hbm, idx_v)
    pltpu.async_copy(feats_hbm.at[idx_v], rows_v, sem).wait()
    pltpu.sync_copy(mu_hbm, mu_v)
    pltpu.sync_copy(isg_hbm, isg_v)

    nch = D_M // 16
    zero = jnp.zeros((16,), jnp.float32)
    pairs = [(b, i2, j2) for b in range(B_SZ)
             for i2 in range(K_REP) for j2 in range(i2 + 1, K_REP)]

    def body(j, carry):
        con = carry[0]
        pacc = carry[1:]
        sl = pl.ds(j * 16, 16)
        row = [[rows_v[b * K_REP + r, sl] for r in range(K_REP)]
               for b in range(B_SZ)]
        for b in range(B_SZ):
            mub = mu_v[b, sl]
            isb = isg_v[b, sl]
            for r in range(K_REP):
                cen = row[b][r] - mub
                con = con + cen * cen * isb
        new_p = []
        for p, (b, i2, j2) in enumerate(pairs):
            new_p.append(pacc[p] + row[b][i2] * row[b][j2])
        return (con,) + tuple(new_p)

    init = (zero,) + tuple(zero for _ in pairs)
    res = lax.fori_loop(0, nch, body, init)

    def sum16(v):
        s = v[0]
        for q in range(1, 16):
            s = s + v[q]
        return s

    con_s = sum16(res[0])
    div_s = jnp.float32(0.0)
    for p in range(len(pairs)):
        d = sum16(res[1 + p])
        div_s = div_s + d * d
    pltpu.sync_copy(ell_hbm, loss_v)
    ell_s = loss_v[...][0]
    loss = (ell_s + 0.1 * (con_s * (1.0 / nk))
            + 0.05 * (div_s * (1.0 / (B_SZ * K_REP * K_REP))))
    loss_v[...] = jnp.broadcast_to(loss, (16,))

    @pl.when(is_lead)
    def _():
        pltpu.sync_copy(rows_v, rep_out)
        pltpu.sync_copy(loss_v, loss_out)


def _sc_gather_loss(feats_flat, gidx, mu_q, inv_sigma, ell):
    nk = B_SZ * K_REP
    run = pl.kernel(
        _sc_body,
        mesh=plsc.VectorSubcoreMesh(core_axis_name="c", subcore_axis_name="s"),
        out_type=[
            jax.ShapeDtypeStruct((16, D_M), jnp.float32),
            jax.ShapeDtypeStruct((16,), jnp.float32),
        ],
        scratch_types=[
            pltpu.VMEM((16,), jnp.int32),
            pltpu.VMEM((16, D_M), jnp.float32),
            pltpu.VMEM((B_SZ, D_M), jnp.float32),
            pltpu.VMEM((B_SZ, D_M), jnp.float32),
            pltpu.VMEM((16,), jnp.float32),
            pltpu.SemaphoreType.DMA,
        ],
    )
    return run(feats_flat, gidx, mu_q, inv_sigma, ell)


def kernel(video_features, query_embedding, mu_w1, mu_b1, mu_g1, mu_be1,
           mu_w2, mu_b2, sg_w1, sg_b1, sg_g1, sg_be1, sg_w2, sg_b2,
           in_proj_w, in_proj_b, out_w, out_b):
    B, T, D = video_features.shape
    NT = T // TB

    def const2(_):
        return (0, 0)

    def const1(_):
        return (0,)

    full2 = pl.BlockSpec((D, D), const2)
    full1 = pl.BlockSpec((D,), const1)

    indices, combined, gidx, ell, mu_q, inv_sigma = pl.pallas_call(
        _fused_kernel,
        grid=(NT + 2,),
        in_specs=[
            pl.BlockSpec((B, TB, D),
                         lambda i: (0, jnp.clip(i - 1, 0, NT - 1), 0)),
            pl.BlockSpec((B, D), const2),
            full2, full1, full1, full1, full2, full1,
            full2, full1, full1, full1, full2, full1,
            pl.BlockSpec((D, D), lambda i: (0, 0)),
            pl.BlockSpec((D,), lambda i: (0,)),
            pl.BlockSpec((D, D), lambda i: (1, 0)),
        ],
        out_specs=(
            pl.BlockSpec(memory_space=pltpu.SMEM),
            pl.BlockSpec((B, T), const2),
            pl.BlockSpec((16,), const1),
            pl.BlockSpec((16,), const1),
            pl.BlockSpec((B, D), const2),
            pl.BlockSpec((B, D), const2),
        ),
        out_shape=(
            jax.ShapeDtypeStruct((B, K_REP), jnp.int32),
            jax.ShapeDtypeStruct((B, T), jnp.float32),
            jax.ShapeDtypeStruct((16,), jnp.int32),
            jax.ShapeDtypeStruct((16,), jnp.float32),
            jax.ShapeDtypeStruct((B, D), jnp.float32),
            jax.ShapeDtypeStruct((B, D), jnp.float32),
        ),
        scratch_shapes=[
            pltpu.VMEM((B, D), jnp.float32),
            pltpu.VMEM((B, D), jnp.float32),
            pltpu.VMEM((B * N_HEADS, D), jnp.float32),
            pltpu.VMEM((B, T), jnp.float32),
            pltpu.VMEM((B * N_HEADS, T), jnp.float32),
        ],
    )(video_features, query_embedding, mu_w1, mu_b1, mu_g1, mu_be1,
      mu_w2, mu_b2, sg_w1, sg_b1, sg_g1, sg_be1, sg_w2, sg_b2,
      in_proj_w, in_proj_b, in_proj_w)

    feats_flat = video_features.reshape(B * T, D)
    rep_flat, loss_vec = _sc_gather_loss(feats_flat, gidx, mu_q, inv_sigma,
                                         ell)
    rep = rep_flat[:B * K_REP].reshape(B, K_REP, D)
    return rep, loss_vec[0], indices, combined
